# Initial kernel scaffold; baseline (speedup 1.0000x reference)
import jax
import jax.numpy as jnp
from jax.experimental import pallas as pl


def kernel(x, edge_index, W1, W2, W3):
    raise NotImplementedError("write your pallas kernel here")


# speedup vs baseline: 5.8561x; 5.8561x over previous
import functools

import jax
import jax.numpy as jnp
from jax import lax
from jax.experimental import pallas as pl
from jax.experimental.pallas import tpu as pltpu
from jax.experimental.pallas import tpu_sc as plsc

N = 10000
NP = 10240
E = 320000
EP = 327680
NC = 2
NS = 16
CHUNK = 256
ROWS_PER_TILE = NP // NS
DEG_PER_TILE = EP // (NC * NS)
DEG_CHUNKS = DEG_PER_TILE // CHUNK
GROUPS = EP // 128
GROUPS_PER_TILE = GROUPS // NS
F32 = jnp.float32

_mesh = plsc.VectorSubcoreMesh(core_axis_name="c", subcore_axis_name="s",
                               num_cores=NC, num_subcores=NS)
_sc_params = pltpu.CompilerParams(use_tc_tiling_on_sc=False)


def _deg_body(dst3d_hbm, ones_hbm, zeros16_hbm, out_hbm,
              idx_v, ones_v, acc, _):
    cid = lax.axis_index("c")
    sid = lax.axis_index("s")
    wid = cid * NS + sid
    pltpu.sync_copy(zeros16_hbm, acc.at[pl.ds(sid * ROWS_PER_TILE, ROWS_PER_TILE)])
    pltpu.sync_copy(ones_hbm, ones_v)
    plsc.subcore_barrier()

    def chunk(i, _):
        pltpu.sync_copy(dst3d_hbm.at[wid * DEG_CHUNKS + i], idx_v)
        for j in range(CHUNK // 128):
            pltpu.sync_copy(ones_v.at[pl.ds(j * 128, 128)],
                            acc.at[idx_v.at[j]], add=True)
        return 0

    lax.fori_loop(0, DEG_CHUNKS, chunk, 0)
    plsc.subcore_barrier()
    sl = pl.ds(sid * ROWS_PER_TILE, ROWS_PER_TILE)

    @pl.when(cid == 0)
    def _():
        pltpu.sync_copy(acc.at[sl], out_hbm.at[0].at[sl])

    @pl.when(cid == 1)
    def _():
        pltpu.sync_copy(acc.at[sl], out_hbm.at[1].at[sl])


_deg_call = functools.partial(
    pl.kernel,
    _deg_body,
    out_type=jax.ShapeDtypeStruct((NC, NP, 16), F32),
    mesh=_mesh,
    compiler_params=_sc_params,
    scratch_types=[
        pltpu.VMEM((CHUNK // 128, 128), jnp.int32),
        pltpu.VMEM((CHUNK, 16), F32),
        pltpu.VMEM_SHARED((NP, 16), F32),
        pltpu.SemaphoreType.DMA,
    ],
)()


def _agg_body(src2d_hbm, dst2d_hbm, g0_hbm, g1_hbm, zeros_hbm,
              out0_hbm, out1_hbm, sidx_v, didx_v, rows_v, acc, sem):
    cid = lax.axis_index("c")
    sid = lax.axis_index("s")
    sl = pl.ds(sid * ROWS_PER_TILE, ROWS_PER_TILE)
    pltpu.sync_copy(zeros_hbm, acc.at[sl])
    plsc.subcore_barrier()

    def run(g_hbm):
        def group(i, _):
            gi = sid * GROUPS_PER_TILE + i
            pltpu.sync_copy(src2d_hbm.at[gi], sidx_v)
            pltpu.sync_copy(dst2d_hbm.at[gi], didx_v)
            pltpu.async_copy(g_hbm.at[sidx_v], rows_v, sem).wait()
            pltpu.sync_copy(rows_v, acc.at[didx_v], add=True)
            return 0

        lax.fori_loop(0, GROUPS_PER_TILE, group, 0)

    @pl.when(cid == 0)
    def _():
        run(g0_hbm)

    @pl.when(cid == 1)
    def _():
        run(g1_hbm)

    plsc.subcore_barrier()

    @pl.when(cid == 0)
    def _():
        pltpu.sync_copy(acc.at[sl], out0_hbm.at[sl])

    @pl.when(cid == 1)
    def _():
        pltpu.sync_copy(acc.at[sl], out1_hbm.at[sl])


_agg_call = functools.partial(
    pl.kernel,
    _agg_body,
    out_type=(jax.ShapeDtypeStruct((NP, 128), F32),
              jax.ShapeDtypeStruct((NP, 128), F32)),
    mesh=_mesh,
    compiler_params=_sc_params,
    scratch_types=[
        pltpu.VMEM((128,), jnp.int32),
        pltpu.VMEM((128,), jnp.int32),
        pltpu.VMEM((128, 128), F32),
        pltpu.VMEM_SHARED((NP, 128), F32),
        pltpu.SemaphoreType.DMA,
    ],
)()


BR = 1024
GRID = NP // BR
_HI = lax.Precision.HIGHEST


def _mm(a, b):
    return lax.dot_general(a, b, (((1,), (0,)), ((), ())),
                           precision=_HI, preferred_element_type=F32)


def _prologue_body(x_ref, w_ref, d0_ref, d1_ref, g0_ref, g1_ref, dinv_ref):
    deg = d0_ref[:, 0:1] + d1_ref[:, 0:1] + 1.0
    dinv = lax.rsqrt(deg)
    g = _mm(x_ref[...], w_ref[...]) * dinv
    g0_ref[...] = g[:, :128]
    g1_ref[...] = g[:, 128:]
    dinv_ref[...] = dinv


_prologue = pl.pallas_call(
    _prologue_body,
    grid=(GRID,),
    in_specs=[
        pl.BlockSpec((BR, 128), lambda i: (i, 0)),
        pl.BlockSpec((128, 256), lambda i: (0, 0)),
        pl.BlockSpec((BR, 16), lambda i: (i, 0)),
        pl.BlockSpec((BR, 16), lambda i: (i, 0)),
    ],
    out_specs=(
        pl.BlockSpec((BR, 128), lambda i: (i, 0)),
        pl.BlockSpec((BR, 128), lambda i: (i, 0)),
        pl.BlockSpec((BR, 1), lambda i: (i, 0)),
    ),
    out_shape=(
        jax.ShapeDtypeStruct((NP, 128), F32),
        jax.ShapeDtypeStruct((NP, 128), F32),
        jax.ShapeDtypeStruct((NP, 1), F32),
    ),
)


def _mid_body(a0_ref, a1_ref, g0_ref, g1_ref, dinv_ref, w_ref,
              ng0_ref, ng1_ref):
    dinv = dinv_ref[...]
    xa = jnp.maximum((a0_ref[...] + g0_ref[...]) * dinv, 0.0)
    xb = jnp.maximum((a1_ref[...] + g1_ref[...]) * dinv, 0.0)
    g = (_mm(xa, w_ref[:128, :]) + _mm(xb, w_ref[128:, :])) * dinv
    ng0_ref[...] = g[:, :128]
    ng1_ref[...] = g[:, 128:]


_mid = pl.pallas_call(
    _mid_body,
    grid=(GRID,),
    in_specs=[
        pl.BlockSpec((BR, 128), lambda i: (i, 0)),
        pl.BlockSpec((BR, 128), lambda i: (i, 0)),
        pl.BlockSpec((BR, 128), lambda i: (i, 0)),
        pl.BlockSpec((BR, 128), lambda i: (i, 0)),
        pl.BlockSpec((BR, 1), lambda i: (i, 0)),
        pl.BlockSpec((256, 256), lambda i: (0, 0)),
    ],
    out_specs=(
        pl.BlockSpec((BR, 128), lambda i: (i, 0)),
        pl.BlockSpec((BR, 128), lambda i: (i, 0)),
    ),
    out_shape=(
        jax.ShapeDtypeStruct((NP, 128), F32),
        jax.ShapeDtypeStruct((NP, 128), F32),
    ),
)


def _final_body(a0_ref, a1_ref, g0_ref, g1_ref, dinv_ref, out_ref):
    dinv = dinv_ref[...]
    out_ref[:, :128] = (a0_ref[...] + g0_ref[...]) * dinv
    out_ref[:, 128:] = (a1_ref[...] + g1_ref[...]) * dinv


_final = pl.pallas_call(
    _final_body,
    grid=(GRID,),
    in_specs=[
        pl.BlockSpec((BR, 128), lambda i: (i, 0)),
        pl.BlockSpec((BR, 128), lambda i: (i, 0)),
        pl.BlockSpec((BR, 128), lambda i: (i, 0)),
        pl.BlockSpec((BR, 128), lambda i: (i, 0)),
        pl.BlockSpec((BR, 1), lambda i: (i, 0)),
    ],
    out_specs=pl.BlockSpec((BR, 256), lambda i: (i, 0)),
    out_shape=jax.ShapeDtypeStruct((NP, 256), F32),
)


def kernel(x, edge_index, W1, W2, W3):
    ei = edge_index.astype(jnp.int32)
    pad = jnp.full((EP - E,), N, dtype=jnp.int32)
    src = jnp.concatenate([ei[0], pad])
    dst = jnp.concatenate([ei[1], pad])
    src2d = src.reshape(GROUPS, 128)
    dst2d = dst.reshape(GROUPS, 128)
    dst3d = dst.reshape(EP // CHUNK, CHUNK // 128, 128)
    xp = jnp.pad(x, ((0, NP - N), (0, 0)))

    ones_c = jnp.ones((CHUNK, 16), F32)
    zeros16 = jnp.zeros((ROWS_PER_TILE, 16), F32)
    zeros128 = jnp.zeros((ROWS_PER_TILE, 128), F32)

    deg_parts = _deg_call(dst3d, ones_c, zeros16)
    g0, g1, dinv = _prologue(xp, W1, deg_parts[0], deg_parts[1])

    a0, a1 = _agg_call(src2d, dst2d, g0, g1, zeros128)
    g0, g1 = _mid(a0, a1, g0, g1, dinv, W2)

    a0, a1 = _agg_call(src2d, dst2d, g0, g1, zeros128)
    g0, g1 = _mid(a0, a1, g0, g1, dinv, W3)

    a0, a1 = _agg_call(src2d, dst2d, g0, g1, zeros128)
    out = _final(a0, a1, g0, g1, dinv)
    return out[:N]


# speedup vs baseline: 6.8236x; 1.1652x over previous
import functools

import jax
import jax.numpy as jnp
from jax import lax
from jax.experimental import pallas as pl
from jax.experimental.pallas import tpu as pltpu
from jax.experimental.pallas import tpu_sc as plsc

N = 10000
NP = 10240
E = 320000
EP = 327680
NC = 2
NS = 16
CHUNK = 256
ROWS_PER_TILE = NP // NS
DEG_PER_TILE = EP // (NC * NS)
DEG_CHUNKS = DEG_PER_TILE // CHUNK
GROUPS = EP // 128
GROUPS_PER_TILE = GROUPS // NS
F32 = jnp.float32

_mesh = plsc.VectorSubcoreMesh(core_axis_name="c", subcore_axis_name="s",
                               num_cores=NC, num_subcores=NS)
_sc_params = pltpu.CompilerParams(use_tc_tiling_on_sc=False)


def _deg_body(dst3d_hbm, ones_hbm, zeros16_hbm, out_hbm,
              idx_v, ones_v, acc, _):
    cid = lax.axis_index("c")
    sid = lax.axis_index("s")
    wid = cid * NS + sid
    pltpu.sync_copy(zeros16_hbm, acc.at[pl.ds(sid * ROWS_PER_TILE, ROWS_PER_TILE)])
    pltpu.sync_copy(ones_hbm, ones_v)
    plsc.subcore_barrier()

    def chunk(i, _):
        pltpu.sync_copy(dst3d_hbm.at[wid * DEG_CHUNKS + i], idx_v)
        for j in range(CHUNK // 128):
            pltpu.sync_copy(ones_v.at[pl.ds(j * 128, 128)],
                            acc.at[idx_v.at[j]], add=True)
        return 0

    lax.fori_loop(0, DEG_CHUNKS, chunk, 0)
    plsc.subcore_barrier()
    sl = pl.ds(sid * ROWS_PER_TILE, ROWS_PER_TILE)

    @pl.when(cid == 0)
    def _():
        pltpu.sync_copy(acc.at[sl], out_hbm.at[0].at[sl])

    @pl.when(cid == 1)
    def _():
        pltpu.sync_copy(acc.at[sl], out_hbm.at[1].at[sl])


_deg_call = functools.partial(
    pl.kernel,
    _deg_body,
    out_type=jax.ShapeDtypeStruct((NC, NP, 16), F32),
    mesh=_mesh,
    compiler_params=_sc_params,
    scratch_types=[
        pltpu.VMEM((CHUNK // 128, 128), jnp.int32),
        pltpu.VMEM((CHUNK, 16), F32),
        pltpu.VMEM_SHARED((NP, 16), F32),
        pltpu.SemaphoreType.DMA,
    ],
)()


NBUF = 2
ITERS = GROUPS_PER_TILE // NBUF


def _agg_body(idx4_hbm, g0_hbm, g1_hbm, zeros_hbm,
              out0_hbm, out1_hbm, idx_v,
              r0, r1, acc, s0, s1):
    rows = [r0, r1]
    sems = [s0, s1]
    cid = lax.axis_index("c")
    sid = lax.axis_index("s")
    sl = pl.ds(sid * ROWS_PER_TILE, ROWS_PER_TILE)
    pltpu.sync_copy(zeros_hbm, acc.at[sl])
    plsc.subcore_barrier()

    def run(g_hbm):
        def body(k, _):
            pltpu.sync_copy(idx4_hbm.at[sid * ITERS + k], idx_v)
            descs = [
                pltpu.async_copy(g_hbm.at[idx_v.at[b].at[0]], rows[b], sems[b])
                for b in range(NBUF)
            ]
            for b in range(NBUF):
                descs[b].wait()
                pltpu.sync_copy(rows[b], acc.at[idx_v.at[b].at[1]], add=True)
            return 0

        lax.fori_loop(0, ITERS, body, 0)

    @pl.when(cid == 0)
    def _():
        run(g0_hbm)

    @pl.when(cid == 1)
    def _():
        run(g1_hbm)

    plsc.subcore_barrier()

    @pl.when(cid == 0)
    def _():
        pltpu.sync_copy(acc.at[sl], out0_hbm.at[sl])

    @pl.when(cid == 1)
    def _():
        pltpu.sync_copy(acc.at[sl], out1_hbm.at[sl])


_agg_call = functools.partial(
    pl.kernel,
    _agg_body,
    out_type=(jax.ShapeDtypeStruct((NP, 128), F32),
              jax.ShapeDtypeStruct((NP, 128), F32)),
    mesh=_mesh,
    compiler_params=_sc_params,
    scratch_types=[
        pltpu.VMEM((NBUF, 2, 128), jnp.int32),
        pltpu.VMEM((128, 128), F32),
        pltpu.VMEM((128, 128), F32),
        pltpu.VMEM_SHARED((NP, 128), F32),
        pltpu.SemaphoreType.DMA,
        pltpu.SemaphoreType.DMA,
    ],
)()


BR = 1024
GRID = NP // BR
_HI = lax.Precision.HIGHEST


def _mm(a, b):
    return lax.dot_general(a, b, (((1,), (0,)), ((), ())),
                           precision=_HI, preferred_element_type=F32)


def _prologue_body(x_ref, w_ref, d0_ref, d1_ref, g0_ref, g1_ref, dinv_ref):
    deg = d0_ref[:, 0:1] + d1_ref[:, 0:1] + 1.0
    dinv = lax.rsqrt(deg)
    g = _mm(x_ref[...], w_ref[...]) * dinv
    g0_ref[...] = g[:, :128]
    g1_ref[...] = g[:, 128:]
    dinv_ref[...] = dinv


_prologue = pl.pallas_call(
    _prologue_body,
    grid=(GRID,),
    in_specs=[
        pl.BlockSpec((BR, 128), lambda i: (i, 0)),
        pl.BlockSpec((128, 256), lambda i: (0, 0)),
        pl.BlockSpec((BR, 16), lambda i: (i, 0)),
        pl.BlockSpec((BR, 16), lambda i: (i, 0)),
    ],
    out_specs=(
        pl.BlockSpec((BR, 128), lambda i: (i, 0)),
        pl.BlockSpec((BR, 128), lambda i: (i, 0)),
        pl.BlockSpec((BR, 1), lambda i: (i, 0)),
    ),
    out_shape=(
        jax.ShapeDtypeStruct((NP, 128), F32),
        jax.ShapeDtypeStruct((NP, 128), F32),
        jax.ShapeDtypeStruct((NP, 1), F32),
    ),
)


def _mid_body(a0_ref, a1_ref, g0_ref, g1_ref, dinv_ref, w_ref,
              ng0_ref, ng1_ref):
    dinv = dinv_ref[...]
    xa = jnp.maximum((a0_ref[...] + g0_ref[...]) * dinv, 0.0)
    xb = jnp.maximum((a1_ref[...] + g1_ref[...]) * dinv, 0.0)
    g = (_mm(xa, w_ref[:128, :]) + _mm(xb, w_ref[128:, :])) * dinv
    ng0_ref[...] = g[:, :128]
    ng1_ref[...] = g[:, 128:]


_mid = pl.pallas_call(
    _mid_body,
    grid=(GRID,),
    in_specs=[
        pl.BlockSpec((BR, 128), lambda i: (i, 0)),
        pl.BlockSpec((BR, 128), lambda i: (i, 0)),
        pl.BlockSpec((BR, 128), lambda i: (i, 0)),
        pl.BlockSpec((BR, 128), lambda i: (i, 0)),
        pl.BlockSpec((BR, 1), lambda i: (i, 0)),
        pl.BlockSpec((256, 256), lambda i: (0, 0)),
    ],
    out_specs=(
        pl.BlockSpec((BR, 128), lambda i: (i, 0)),
        pl.BlockSpec((BR, 128), lambda i: (i, 0)),
    ),
    out_shape=(
        jax.ShapeDtypeStruct((NP, 128), F32),
        jax.ShapeDtypeStruct((NP, 128), F32),
    ),
)


def _final_body(a0_ref, a1_ref, g0_ref, g1_ref, dinv_ref, out_ref):
    dinv = dinv_ref[...]
    out_ref[:, :128] = (a0_ref[...] + g0_ref[...]) * dinv
    out_ref[:, 128:] = (a1_ref[...] + g1_ref[...]) * dinv


_final = pl.pallas_call(
    _final_body,
    grid=(GRID,),
    in_specs=[
        pl.BlockSpec((BR, 128), lambda i: (i, 0)),
        pl.BlockSpec((BR, 128), lambda i: (i, 0)),
        pl.BlockSpec((BR, 128), lambda i: (i, 0)),
        pl.BlockSpec((BR, 128), lambda i: (i, 0)),
        pl.BlockSpec((BR, 1), lambda i: (i, 0)),
    ],
    out_specs=pl.BlockSpec((BR, 256), lambda i: (i, 0)),
    out_shape=jax.ShapeDtypeStruct((NP, 256), F32),
)


def kernel(x, edge_index, W1, W2, W3):
    ei = edge_index.astype(jnp.int32)
    pad = jnp.full((EP - E,), N, dtype=jnp.int32)
    src = jnp.concatenate([ei[0], pad])
    dst = jnp.concatenate([ei[1], pad])
    idx4 = jnp.stack([src.reshape(GROUPS, 128), dst.reshape(GROUPS, 128)],
                     axis=1).reshape(GROUPS // NBUF, NBUF, 2, 128)
    dst3d = dst.reshape(EP // CHUNK, CHUNK // 128, 128)
    xp = jnp.pad(x, ((0, NP - N), (0, 0)))

    ones_c = jnp.ones((CHUNK, 16), F32)
    zeros16 = jnp.zeros((ROWS_PER_TILE, 16), F32)
    zeros128 = jnp.zeros((ROWS_PER_TILE, 128), F32)

    deg_parts = _deg_call(dst3d, ones_c, zeros16)
    g0, g1, dinv = _prologue(xp, W1, deg_parts[0], deg_parts[1])

    a0, a1 = _agg_call(idx4, g0, g1, zeros128)
    g0, g1 = _mid(a0, a1, g0, g1, dinv, W2)

    a0, a1 = _agg_call(idx4, g0, g1, zeros128)
    g0, g1 = _mid(a0, a1, g0, g1, dinv, W3)

    a0, a1 = _agg_call(idx4, g0, g1, zeros128)
    out = _final(a0, a1, g0, g1, dinv)
    return out[:N]


# speedup vs baseline: 7.5862x; 1.1118x over previous
import functools

import jax
import jax.numpy as jnp
from jax import lax
from jax.experimental import pallas as pl
from jax.experimental.pallas import tpu as pltpu
from jax.experimental.pallas import tpu_sc as plsc

N = 10000
NP = 10240
E = 320000
EP = 327680
NC = 2
NS = 16
CHUNK = 256
ROWS_PER_TILE = NP // NS
DEG_PER_TILE = EP // (NC * NS)
DEG_CHUNKS = DEG_PER_TILE // CHUNK
GROUPS = EP // 128
GROUPS_PER_TILE = GROUPS // NS
F32 = jnp.float32

_mesh = plsc.VectorSubcoreMesh(core_axis_name="c", subcore_axis_name="s",
                               num_cores=NC, num_subcores=NS)
_sc_params = pltpu.CompilerParams(use_tc_tiling_on_sc=False)


def _deg_body(dst3d_hbm, ones_hbm, zeros16_hbm, out_hbm,
              idx_v, ones_v, acc, _):
    cid = lax.axis_index("c")
    sid = lax.axis_index("s")
    wid = cid * NS + sid
    pltpu.sync_copy(zeros16_hbm, acc.at[pl.ds(sid * ROWS_PER_TILE, ROWS_PER_TILE)])
    pltpu.sync_copy(ones_hbm, ones_v)
    plsc.subcore_barrier()

    def chunk(i, _):
        pltpu.sync_copy(dst3d_hbm.at[wid * DEG_CHUNKS + i], idx_v)
        for j in range(CHUNK // 128):
            pltpu.sync_copy(ones_v.at[pl.ds(j * 128, 128)],
                            acc.at[idx_v.at[j]], add=True)
        return 0

    lax.fori_loop(0, DEG_CHUNKS, chunk, 0)
    plsc.subcore_barrier()
    sl = pl.ds(sid * ROWS_PER_TILE, ROWS_PER_TILE)

    @pl.when(cid == 0)
    def _():
        pltpu.sync_copy(acc.at[sl], out_hbm.at[0].at[sl])

    @pl.when(cid == 1)
    def _():
        pltpu.sync_copy(acc.at[sl], out_hbm.at[1].at[sl])


_deg_call = functools.partial(
    pl.kernel,
    _deg_body,
    out_type=jax.ShapeDtypeStruct((NC, NP, 16), F32),
    mesh=_mesh,
    compiler_params=_sc_params,
    scratch_types=[
        pltpu.VMEM((CHUNK // 128, 128), jnp.int32),
        pltpu.VMEM((CHUNK, 16), F32),
        pltpu.VMEM_SHARED((NP, 16), F32),
        pltpu.SemaphoreType.DMA,
    ],
)()


NBUF = 2
ITERS = GROUPS_PER_TILE // NBUF


def _agg_body(idx4_hbm, g0_hbm, g1_hbm, zeros_hbm,
              out0_hbm, out1_hbm, idxA, idxB,
              r0, r1, acc, g0s, g1s, s0s, s1s):
    rows = [r0, r1]
    gsem = [g0s, g1s]
    ssem = [s0s, s1s]
    cid = lax.axis_index("c")
    sid = lax.axis_index("s")
    sl = pl.ds(sid * ROWS_PER_TILE, ROWS_PER_TILE)
    pltpu.sync_copy(zeros_hbm, acc.at[sl])
    plsc.subcore_barrier()

    def run(g_hbm):
        base = sid * ITERS
        pltpu.sync_copy(idx4_hbm.at[base], idxA)
        for b in range(NBUF):
            pltpu.async_copy(g_hbm.at[idxA.at[b].at[0]], rows[b], gsem[b])

        def phase(t_next, idx_cur, idx_nxt):
            @pl.when(t_next < ITERS)
            def _():
                pltpu.sync_copy(idx4_hbm.at[base + t_next], idx_nxt)

            for b in range(NBUF):
                pltpu.make_async_copy(g_hbm.at[idx_cur.at[b].at[0]],
                                      rows[b], gsem[b]).wait()
                pltpu.async_copy(rows[b], acc.at[idx_cur.at[b].at[1]],
                                 ssem[b], add=True)
            for b in range(NBUF):
                pltpu.make_async_copy(rows[b], acc.at[idx_cur.at[b].at[1]],
                                      ssem[b]).wait()

                @pl.when(t_next < ITERS)
                def _():
                    pltpu.async_copy(g_hbm.at[idx_nxt.at[b].at[0]],
                                     rows[b], gsem[b])

        def body(u, _):
            phase(2 * u + 1, idxA, idxB)
            phase(2 * u + 2, idxB, idxA)
            return 0

        lax.fori_loop(0, ITERS // 2, body, 0)

    @pl.when(cid == 0)
    def _():
        run(g0_hbm)

    @pl.when(cid == 1)
    def _():
        run(g1_hbm)

    plsc.subcore_barrier()

    @pl.when(cid == 0)
    def _():
        pltpu.sync_copy(acc.at[sl], out0_hbm.at[sl])

    @pl.when(cid == 1)
    def _():
        pltpu.sync_copy(acc.at[sl], out1_hbm.at[sl])


_agg_call = functools.partial(
    pl.kernel,
    _agg_body,
    out_type=(jax.ShapeDtypeStruct((NP, 128), F32),
              jax.ShapeDtypeStruct((NP, 128), F32)),
    mesh=_mesh,
    compiler_params=_sc_params,
    scratch_types=[
        pltpu.VMEM((NBUF, 2, 128), jnp.int32),
        pltpu.VMEM((NBUF, 2, 128), jnp.int32),
        pltpu.VMEM((128, 128), F32),
        pltpu.VMEM((128, 128), F32),
        pltpu.VMEM_SHARED((NP, 128), F32),
        pltpu.SemaphoreType.DMA,
        pltpu.SemaphoreType.DMA,
        pltpu.SemaphoreType.DMA,
        pltpu.SemaphoreType.DMA,
    ],
)()


BR = 1024
GRID = NP // BR
_HI = lax.Precision.HIGHEST


def _mm(a, b):
    return lax.dot_general(a, b, (((1,), (0,)), ((), ())),
                           precision=_HI, preferred_element_type=F32)


def _prologue_body(x_ref, w_ref, d0_ref, d1_ref, g0_ref, g1_ref, dinv_ref):
    deg = d0_ref[:, 0:1] + d1_ref[:, 0:1] + 1.0
    dinv = lax.rsqrt(deg)
    g = _mm(x_ref[...], w_ref[...]) * dinv
    g0_ref[...] = g[:, :128]
    g1_ref[...] = g[:, 128:]
    dinv_ref[...] = dinv


_prologue = pl.pallas_call(
    _prologue_body,
    grid=(GRID,),
    in_specs=[
        pl.BlockSpec((BR, 128), lambda i: (i, 0)),
        pl.BlockSpec((128, 256), lambda i: (0, 0)),
        pl.BlockSpec((BR, 16), lambda i: (i, 0)),
        pl.BlockSpec((BR, 16), lambda i: (i, 0)),
    ],
    out_specs=(
        pl.BlockSpec((BR, 128), lambda i: (i, 0)),
        pl.BlockSpec((BR, 128), lambda i: (i, 0)),
        pl.BlockSpec((BR, 1), lambda i: (i, 0)),
    ),
    out_shape=(
        jax.ShapeDtypeStruct((NP, 128), F32),
        jax.ShapeDtypeStruct((NP, 128), F32),
        jax.ShapeDtypeStruct((NP, 1), F32),
    ),
)


def _mid_body(a0_ref, a1_ref, g0_ref, g1_ref, dinv_ref, w_ref,
              ng0_ref, ng1_ref):
    dinv = dinv_ref[...]
    xa = jnp.maximum((a0_ref[...] + g0_ref[...]) * dinv, 0.0)
    xb = jnp.maximum((a1_ref[...] + g1_ref[...]) * dinv, 0.0)
    g = (_mm(xa, w_ref[:128, :]) + _mm(xb, w_ref[128:, :])) * dinv
    ng0_ref[...] = g[:, :128]
    ng1_ref[...] = g[:, 128:]


_mid = pl.pallas_call(
    _mid_body,
    grid=(GRID,),
    in_specs=[
        pl.BlockSpec((BR, 128), lambda i: (i, 0)),
        pl.BlockSpec((BR, 128), lambda i: (i, 0)),
        pl.BlockSpec((BR, 128), lambda i: (i, 0)),
        pl.BlockSpec((BR, 128), lambda i: (i, 0)),
        pl.BlockSpec((BR, 1), lambda i: (i, 0)),
        pl.BlockSpec((256, 256), lambda i: (0, 0)),
    ],
    out_specs=(
        pl.BlockSpec((BR, 128), lambda i: (i, 0)),
        pl.BlockSpec((BR, 128), lambda i: (i, 0)),
    ),
    out_shape=(
        jax.ShapeDtypeStruct((NP, 128), F32),
        jax.ShapeDtypeStruct((NP, 128), F32),
    ),
)


def _final_body(a0_ref, a1_ref, g0_ref, g1_ref, dinv_ref, out_ref):
    dinv = dinv_ref[...]
    out_ref[:, :128] = (a0_ref[...] + g0_ref[...]) * dinv
    out_ref[:, 128:] = (a1_ref[...] + g1_ref[...]) * dinv


_final = pl.pallas_call(
    _final_body,
    grid=(GRID,),
    in_specs=[
        pl.BlockSpec((BR, 128), lambda i: (i, 0)),
        pl.BlockSpec((BR, 128), lambda i: (i, 0)),
        pl.BlockSpec((BR, 128), lambda i: (i, 0)),
        pl.BlockSpec((BR, 128), lambda i: (i, 0)),
        pl.BlockSpec((BR, 1), lambda i: (i, 0)),
    ],
    out_specs=pl.BlockSpec((BR, 256), lambda i: (i, 0)),
    out_shape=jax.ShapeDtypeStruct((NP, 256), F32),
)


def kernel(x, edge_index, W1, W2, W3):
    ei = edge_index.astype(jnp.int32)
    pad = jnp.full((EP - E,), N, dtype=jnp.int32)
    src = jnp.concatenate([ei[0], pad])
    dst = jnp.concatenate([ei[1], pad])
    idx4 = jnp.stack([src.reshape(GROUPS, 128), dst.reshape(GROUPS, 128)],
                     axis=1).reshape(GROUPS // NBUF, NBUF, 2, 128)
    dst3d = dst.reshape(EP // CHUNK, CHUNK // 128, 128)
    xp = jnp.pad(x, ((0, NP - N), (0, 0)))

    ones_c = jnp.ones((CHUNK, 16), F32)
    zeros16 = jnp.zeros((ROWS_PER_TILE, 16), F32)
    zeros128 = jnp.zeros((ROWS_PER_TILE, 128), F32)

    deg_parts = _deg_call(dst3d, ones_c, zeros16)
    g0, g1, dinv = _prologue(xp, W1, deg_parts[0], deg_parts[1])

    a0, a1 = _agg_call(idx4, g0, g1, zeros128)
    g0, g1 = _mid(a0, a1, g0, g1, dinv, W2)

    a0, a1 = _agg_call(idx4, g0, g1, zeros128)
    g0, g1 = _mid(a0, a1, g0, g1, dinv, W3)

    a0, a1 = _agg_call(idx4, g0, g1, zeros128)
    out = _final(a0, a1, g0, g1, dinv)
    return out[:N]


# speedup vs baseline: 7.9542x; 1.0485x over previous
import functools

import jax
import jax.numpy as jnp
from jax import lax
from jax.experimental import pallas as pl
from jax.experimental.pallas import tpu as pltpu
from jax.experimental.pallas import tpu_sc as plsc

N = 10000
NP = 10240
E = 320000
EP = 327680
NC = 2
NS = 16
CHUNK = 256
ROWS_PER_TILE = NP // NS
DEG_PER_TILE = EP // (NC * NS)
DEG_CHUNKS = DEG_PER_TILE // CHUNK
GW = 64
GROUPS = EP // GW
GROUPS_PER_TILE = GROUPS // NS
F32 = jnp.float32

_mesh = plsc.VectorSubcoreMesh(core_axis_name="c", subcore_axis_name="s",
                               num_cores=NC, num_subcores=NS)
_sc_params = pltpu.CompilerParams(use_tc_tiling_on_sc=False)


def _deg_body(dst3d_hbm, ones_hbm, zeros16_hbm, out_hbm,
              idx_v, ones_v, acc, _):
    cid = lax.axis_index("c")
    sid = lax.axis_index("s")
    wid = cid * NS + sid
    pltpu.sync_copy(zeros16_hbm, acc.at[pl.ds(sid * ROWS_PER_TILE, ROWS_PER_TILE)])
    pltpu.sync_copy(ones_hbm, ones_v)
    plsc.subcore_barrier()

    def chunk(i, _):
        pltpu.sync_copy(dst3d_hbm.at[wid * DEG_CHUNKS + i], idx_v)
        for j in range(CHUNK // 128):
            pltpu.sync_copy(ones_v.at[pl.ds(j * 128, 128)],
                            acc.at[idx_v.at[j]], add=True)
        return 0

    lax.fori_loop(0, DEG_CHUNKS, chunk, 0)
    plsc.subcore_barrier()
    sl = pl.ds(sid * ROWS_PER_TILE, ROWS_PER_TILE)

    @pl.when(cid == 0)
    def _():
        pltpu.sync_copy(acc.at[sl], out_hbm.at[0].at[sl])

    @pl.when(cid == 1)
    def _():
        pltpu.sync_copy(acc.at[sl], out_hbm.at[1].at[sl])


_deg_call = functools.partial(
    pl.kernel,
    _deg_body,
    out_type=jax.ShapeDtypeStruct((NC, NP, 16), F32),
    mesh=_mesh,
    compiler_params=_sc_params,
    scratch_types=[
        pltpu.VMEM((CHUNK // 128, 128), jnp.int32),
        pltpu.VMEM((CHUNK, 16), F32),
        pltpu.VMEM_SHARED((NP, 16), F32),
        pltpu.SemaphoreType.DMA,
    ],
)()


NBUF = 4
ITERS = GROUPS_PER_TILE // NBUF


def _agg_body(idx4_hbm, g0_hbm, g1_hbm, zeros_hbm,
              out0_hbm, out1_hbm, idxA, idxB,
              r0, r1, r2, r3, acc, g0s, g1s, g2s, g3s, s0s, s1s, s2s, s3s):
    rows = [r0, r1, r2, r3]
    gsem = [g0s, g1s, g2s, g3s]
    ssem = [s0s, s1s, s2s, s3s]
    cid = lax.axis_index("c")
    sid = lax.axis_index("s")
    sl = pl.ds(sid * ROWS_PER_TILE, ROWS_PER_TILE)
    pltpu.sync_copy(zeros_hbm, acc.at[sl])
    plsc.subcore_barrier()

    def run(g_hbm):
        base = sid * ITERS
        pltpu.sync_copy(idx4_hbm.at[base], idxA)
        for b in range(NBUF):
            pltpu.async_copy(g_hbm.at[idxA.at[b].at[0]], rows[b], gsem[b])

        def phase(t_next, idx_cur, idx_nxt):
            @pl.when(t_next < ITERS)
            def _():
                pltpu.sync_copy(idx4_hbm.at[base + t_next], idx_nxt)

            for b in range(NBUF):
                pltpu.make_async_copy(g_hbm.at[idx_cur.at[b].at[0]],
                                      rows[b], gsem[b]).wait()
                pltpu.async_copy(rows[b], acc.at[idx_cur.at[b].at[1]],
                                 ssem[b], add=True)
            for b in range(NBUF):
                pltpu.make_async_copy(rows[b], acc.at[idx_cur.at[b].at[1]],
                                      ssem[b]).wait()

                @pl.when(t_next < ITERS)
                def _():
                    pltpu.async_copy(g_hbm.at[idx_nxt.at[b].at[0]],
                                     rows[b], gsem[b])

        def body(u, _):
            phase(2 * u + 1, idxA, idxB)
            phase(2 * u + 2, idxB, idxA)
            return 0

        lax.fori_loop(0, ITERS // 2, body, 0)

    @pl.when(cid == 0)
    def _():
        run(g0_hbm)

    @pl.when(cid == 1)
    def _():
        run(g1_hbm)

    plsc.subcore_barrier()

    @pl.when(cid == 0)
    def _():
        pltpu.sync_copy(acc.at[sl], out0_hbm.at[sl])

    @pl.when(cid == 1)
    def _():
        pltpu.sync_copy(acc.at[sl], out1_hbm.at[sl])


_agg_call = functools.partial(
    pl.kernel,
    _agg_body,
    out_type=(jax.ShapeDtypeStruct((NP, 128), F32),
              jax.ShapeDtypeStruct((NP, 128), F32)),
    mesh=_mesh,
    compiler_params=_sc_params,
    scratch_types=[
        pltpu.VMEM((NBUF, 2, GW), jnp.int32),
        pltpu.VMEM((NBUF, 2, GW), jnp.int32),
        pltpu.VMEM((GW, 128), F32),
        pltpu.VMEM((GW, 128), F32),
        pltpu.VMEM((GW, 128), F32),
        pltpu.VMEM((GW, 128), F32),
        pltpu.VMEM_SHARED((NP, 128), F32),
        pltpu.SemaphoreType.DMA,
        pltpu.SemaphoreType.DMA,
        pltpu.SemaphoreType.DMA,
        pltpu.SemaphoreType.DMA,
        pltpu.SemaphoreType.DMA,
        pltpu.SemaphoreType.DMA,
        pltpu.SemaphoreType.DMA,
        pltpu.SemaphoreType.DMA,
    ],
)()


BR = 1024
GRID = NP // BR
_HI = lax.Precision.HIGHEST


def _mm(a, b):
    return lax.dot_general(a, b, (((1,), (0,)), ((), ())),
                           precision=_HI, preferred_element_type=F32)


def _prologue_body(x_ref, w_ref, d0_ref, d1_ref, g0_ref, g1_ref, dinv_ref):
    deg = d0_ref[:, 0:1] + d1_ref[:, 0:1] + 1.0
    dinv = lax.rsqrt(deg)
    g = _mm(x_ref[...], w_ref[...]) * dinv
    g0_ref[...] = g[:, :128]
    g1_ref[...] = g[:, 128:]
    dinv_ref[...] = dinv


_prologue = pl.pallas_call(
    _prologue_body,
    grid=(GRID,),
    in_specs=[
        pl.BlockSpec((BR, 128), lambda i: (i, 0)),
        pl.BlockSpec((128, 256), lambda i: (0, 0)),
        pl.BlockSpec((BR, 16), lambda i: (i, 0)),
        pl.BlockSpec((BR, 16), lambda i: (i, 0)),
    ],
    out_specs=(
        pl.BlockSpec((BR, 128), lambda i: (i, 0)),
        pl.BlockSpec((BR, 128), lambda i: (i, 0)),
        pl.BlockSpec((BR, 1), lambda i: (i, 0)),
    ),
    out_shape=(
        jax.ShapeDtypeStruct((NP, 128), F32),
        jax.ShapeDtypeStruct((NP, 128), F32),
        jax.ShapeDtypeStruct((NP, 1), F32),
    ),
)


def _mid_body(a0_ref, a1_ref, g0_ref, g1_ref, dinv_ref, w_ref,
              ng0_ref, ng1_ref):
    dinv = dinv_ref[...]
    xa = jnp.maximum((a0_ref[...] + g0_ref[...]) * dinv, 0.0)
    xb = jnp.maximum((a1_ref[...] + g1_ref[...]) * dinv, 0.0)
    g = (_mm(xa, w_ref[:128, :]) + _mm(xb, w_ref[128:, :])) * dinv
    ng0_ref[...] = g[:, :128]
    ng1_ref[...] = g[:, 128:]


_mid = pl.pallas_call(
    _mid_body,
    grid=(GRID,),
    in_specs=[
        pl.BlockSpec((BR, 128), lambda i: (i, 0)),
        pl.BlockSpec((BR, 128), lambda i: (i, 0)),
        pl.BlockSpec((BR, 128), lambda i: (i, 0)),
        pl.BlockSpec((BR, 128), lambda i: (i, 0)),
        pl.BlockSpec((BR, 1), lambda i: (i, 0)),
        pl.BlockSpec((256, 256), lambda i: (0, 0)),
    ],
    out_specs=(
        pl.BlockSpec((BR, 128), lambda i: (i, 0)),
        pl.BlockSpec((BR, 128), lambda i: (i, 0)),
    ),
    out_shape=(
        jax.ShapeDtypeStruct((NP, 128), F32),
        jax.ShapeDtypeStruct((NP, 128), F32),
    ),
)


def _final_body(a0_ref, a1_ref, g0_ref, g1_ref, dinv_ref, out_ref):
    dinv = dinv_ref[...]
    out_ref[:, :128] = (a0_ref[...] + g0_ref[...]) * dinv
    out_ref[:, 128:] = (a1_ref[...] + g1_ref[...]) * dinv


_final = pl.pallas_call(
    _final_body,
    grid=(GRID,),
    in_specs=[
        pl.BlockSpec((BR, 128), lambda i: (i, 0)),
        pl.BlockSpec((BR, 128), lambda i: (i, 0)),
        pl.BlockSpec((BR, 128), lambda i: (i, 0)),
        pl.BlockSpec((BR, 128), lambda i: (i, 0)),
        pl.BlockSpec((BR, 1), lambda i: (i, 0)),
    ],
    out_specs=pl.BlockSpec((BR, 256), lambda i: (i, 0)),
    out_shape=jax.ShapeDtypeStruct((NP, 256), F32),
)


def kernel(x, edge_index, W1, W2, W3):
    ei = edge_index.astype(jnp.int32)
    pad = jnp.full((EP - E,), N, dtype=jnp.int32)
    src = jnp.concatenate([ei[0], pad])
    dst = jnp.concatenate([ei[1], pad])
    idx4 = jnp.stack([src.reshape(GROUPS, GW), dst.reshape(GROUPS, GW)],
                     axis=1).reshape(GROUPS // NBUF, NBUF, 2, GW)
    dst3d = dst.reshape(EP // CHUNK, CHUNK // 128, 128)
    xp = jnp.pad(x, ((0, NP - N), (0, 0)))

    ones_c = jnp.ones((CHUNK, 16), F32)
    zeros16 = jnp.zeros((ROWS_PER_TILE, 16), F32)
    zeros128 = jnp.zeros((ROWS_PER_TILE, 128), F32)

    deg_parts = _deg_call(dst3d, ones_c, zeros16)
    g0, g1, dinv = _prologue(xp, W1, deg_parts[0], deg_parts[1])

    a0, a1 = _agg_call(idx4, g0, g1, zeros128)
    g0, g1 = _mid(a0, a1, g0, g1, dinv, W2)

    a0, a1 = _agg_call(idx4, g0, g1, zeros128)
    g0, g1 = _mid(a0, a1, g0, g1, dinv, W3)

    a0, a1 = _agg_call(idx4, g0, g1, zeros128)
    out = _final(a0, a1, g0, g1, dinv)
    return out[:N]


# speedup vs baseline: 8.0160x; 1.0078x over previous
import functools

import jax
import jax.numpy as jnp
from jax import lax
from jax.experimental import pallas as pl
from jax.experimental.pallas import tpu as pltpu
from jax.experimental.pallas import tpu_sc as plsc

N = 10000
NP = 10240
E = 320000
EP = 327680
NC = 2
NS = 16
CHUNK = 256
ROWS_PER_TILE = NP // NS
DEG_PER_TILE = EP // (NC * NS)
DEG_CHUNKS = DEG_PER_TILE // CHUNK
GW = 32
GROUPS = EP // GW
GROUPS_PER_TILE = GROUPS // NS
F32 = jnp.float32

_mesh = plsc.VectorSubcoreMesh(core_axis_name="c", subcore_axis_name="s",
                               num_cores=NC, num_subcores=NS)
_sc_params = pltpu.CompilerParams(use_tc_tiling_on_sc=False)


def _deg_body(dst3d_hbm, ones_hbm, zeros16_hbm, out_hbm,
              idx_v, ones_v, acc, _):
    cid = lax.axis_index("c")
    sid = lax.axis_index("s")
    wid = cid * NS + sid
    pltpu.sync_copy(zeros16_hbm, acc.at[pl.ds(sid * ROWS_PER_TILE, ROWS_PER_TILE)])
    pltpu.sync_copy(ones_hbm, ones_v)
    plsc.subcore_barrier()

    def chunk(i, _):
        pltpu.sync_copy(dst3d_hbm.at[wid * DEG_CHUNKS + i], idx_v)
        for j in range(CHUNK // 128):
            pltpu.sync_copy(ones_v.at[pl.ds(j * 128, 128)],
                            acc.at[idx_v.at[j]], add=True)
        return 0

    lax.fori_loop(0, DEG_CHUNKS, chunk, 0)
    plsc.subcore_barrier()
    sl = pl.ds(sid * ROWS_PER_TILE, ROWS_PER_TILE)

    @pl.when(cid == 0)
    def _():
        pltpu.sync_copy(acc.at[sl], out_hbm.at[0].at[sl])

    @pl.when(cid == 1)
    def _():
        pltpu.sync_copy(acc.at[sl], out_hbm.at[1].at[sl])


_deg_call = functools.partial(
    pl.kernel,
    _deg_body,
    out_type=jax.ShapeDtypeStruct((NC, NP, 16), F32),
    mesh=_mesh,
    compiler_params=_sc_params,
    scratch_types=[
        pltpu.VMEM((CHUNK // 128, 128), jnp.int32),
        pltpu.VMEM((CHUNK, 16), F32),
        pltpu.VMEM_SHARED((NP, 16), F32),
        pltpu.SemaphoreType.DMA,
    ],
)()


NBUF = 8
ITERS = GROUPS_PER_TILE // NBUF


def _agg_body(idx4_hbm, g0_hbm, g1_hbm, zeros_hbm,
              out0_hbm, out1_hbm, idxA, idxB,
              r0, r1, r2, r3, r4, r5, r6, r7, acc,
              g0s, g1s, g2s, g3s, g4s, g5s, g6s, g7s,
              s0s, s1s, s2s, s3s, s4s, s5s, s6s, s7s):
    rows = [r0, r1, r2, r3, r4, r5, r6, r7]
    gsem = [g0s, g1s, g2s, g3s, g4s, g5s, g6s, g7s]
    ssem = [s0s, s1s, s2s, s3s, s4s, s5s, s6s, s7s]
    cid = lax.axis_index("c")
    sid = lax.axis_index("s")
    sl = pl.ds(sid * ROWS_PER_TILE, ROWS_PER_TILE)
    pltpu.sync_copy(zeros_hbm, acc.at[sl])
    plsc.subcore_barrier()

    def run(g_hbm):
        base = sid * ITERS
        pltpu.sync_copy(idx4_hbm.at[base], idxA)
        for b in range(NBUF):
            pltpu.async_copy(g_hbm.at[idxA.at[b].at[0]], rows[b], gsem[b])

        def phase(t_next, idx_cur, idx_nxt):
            @pl.when(t_next < ITERS)
            def _():
                pltpu.sync_copy(idx4_hbm.at[base + t_next], idx_nxt)

            for b in range(NBUF):
                pltpu.make_async_copy(g_hbm.at[idx_cur.at[b].at[0]],
                                      rows[b], gsem[b]).wait()
                pltpu.async_copy(rows[b], acc.at[idx_cur.at[b].at[1]],
                                 ssem[b], add=True)
            for b in range(NBUF):
                pltpu.make_async_copy(rows[b], acc.at[idx_cur.at[b].at[1]],
                                      ssem[b]).wait()

                @pl.when(t_next < ITERS)
                def _():
                    pltpu.async_copy(g_hbm.at[idx_nxt.at[b].at[0]],
                                     rows[b], gsem[b])

        def body(u, _):
            phase(2 * u + 1, idxA, idxB)
            phase(2 * u + 2, idxB, idxA)
            return 0

        lax.fori_loop(0, ITERS // 2, body, 0)

    @pl.when(cid == 0)
    def _():
        run(g0_hbm)

    @pl.when(cid == 1)
    def _():
        run(g1_hbm)

    plsc.subcore_barrier()

    @pl.when(cid == 0)
    def _():
        pltpu.sync_copy(acc.at[sl], out0_hbm.at[sl])

    @pl.when(cid == 1)
    def _():
        pltpu.sync_copy(acc.at[sl], out1_hbm.at[sl])


_agg_call = functools.partial(
    pl.kernel,
    _agg_body,
    out_type=(jax.ShapeDtypeStruct((NP, 128), F32),
              jax.ShapeDtypeStruct((NP, 128), F32)),
    mesh=_mesh,
    compiler_params=_sc_params,
    scratch_types=[
        pltpu.VMEM((NBUF, 2, GW), jnp.int32),
        pltpu.VMEM((NBUF, 2, GW), jnp.int32),
        pltpu.VMEM((GW, 128), F32),
        pltpu.VMEM((GW, 128), F32),
        pltpu.VMEM((GW, 128), F32),
        pltpu.VMEM((GW, 128), F32),
        pltpu.VMEM((GW, 128), F32),
        pltpu.VMEM((GW, 128), F32),
        pltpu.VMEM((GW, 128), F32),
        pltpu.VMEM((GW, 128), F32),
        pltpu.VMEM_SHARED((NP, 128), F32),
        pltpu.SemaphoreType.DMA,
        pltpu.SemaphoreType.DMA,
        pltpu.SemaphoreType.DMA,
        pltpu.SemaphoreType.DMA,
        pltpu.SemaphoreType.DMA,
        pltpu.SemaphoreType.DMA,
        pltpu.SemaphoreType.DMA,
        pltpu.SemaphoreType.DMA,
        pltpu.SemaphoreType.DMA,
        pltpu.SemaphoreType.DMA,
        pltpu.SemaphoreType.DMA,
        pltpu.SemaphoreType.DMA,
        pltpu.SemaphoreType.DMA,
        pltpu.SemaphoreType.DMA,
        pltpu.SemaphoreType.DMA,
        pltpu.SemaphoreType.DMA,
    ],
)()


BR = 1024
GRID = NP // BR
_HI = lax.Precision.HIGHEST


def _mm(a, b):
    return lax.dot_general(a, b, (((1,), (0,)), ((), ())),
                           precision=_HI, preferred_element_type=F32)


def _prologue_body(x_ref, w_ref, d0_ref, d1_ref, g0_ref, g1_ref, dinv_ref):
    deg = d0_ref[:, 0:1] + d1_ref[:, 0:1] + 1.0
    dinv = lax.rsqrt(deg)
    g = _mm(x_ref[...], w_ref[...]) * dinv
    g0_ref[...] = g[:, :128]
    g1_ref[...] = g[:, 128:]
    dinv_ref[...] = dinv


_prologue = pl.pallas_call(
    _prologue_body,
    grid=(GRID,),
    in_specs=[
        pl.BlockSpec((BR, 128), lambda i: (i, 0)),
        pl.BlockSpec((128, 256), lambda i: (0, 0)),
        pl.BlockSpec((BR, 16), lambda i: (i, 0)),
        pl.BlockSpec((BR, 16), lambda i: (i, 0)),
    ],
    out_specs=(
        pl.BlockSpec((BR, 128), lambda i: (i, 0)),
        pl.BlockSpec((BR, 128), lambda i: (i, 0)),
        pl.BlockSpec((BR, 1), lambda i: (i, 0)),
    ),
    out_shape=(
        jax.ShapeDtypeStruct((NP, 128), F32),
        jax.ShapeDtypeStruct((NP, 128), F32),
        jax.ShapeDtypeStruct((NP, 1), F32),
    ),
)


def _mid_body(a0_ref, a1_ref, g0_ref, g1_ref, dinv_ref, w_ref,
              ng0_ref, ng1_ref):
    dinv = dinv_ref[...]
    xa = jnp.maximum((a0_ref[...] + g0_ref[...]) * dinv, 0.0)
    xb = jnp.maximum((a1_ref[...] + g1_ref[...]) * dinv, 0.0)
    g = (_mm(xa, w_ref[:128, :]) + _mm(xb, w_ref[128:, :])) * dinv
    ng0_ref[...] = g[:, :128]
    ng1_ref[...] = g[:, 128:]


_mid = pl.pallas_call(
    _mid_body,
    grid=(GRID,),
    in_specs=[
        pl.BlockSpec((BR, 128), lambda i: (i, 0)),
        pl.BlockSpec((BR, 128), lambda i: (i, 0)),
        pl.BlockSpec((BR, 128), lambda i: (i, 0)),
        pl.BlockSpec((BR, 128), lambda i: (i, 0)),
        pl.BlockSpec((BR, 1), lambda i: (i, 0)),
        pl.BlockSpec((256, 256), lambda i: (0, 0)),
    ],
    out_specs=(
        pl.BlockSpec((BR, 128), lambda i: (i, 0)),
        pl.BlockSpec((BR, 128), lambda i: (i, 0)),
    ),
    out_shape=(
        jax.ShapeDtypeStruct((NP, 128), F32),
        jax.ShapeDtypeStruct((NP, 128), F32),
    ),
)


def _final_body(a0_ref, a1_ref, g0_ref, g1_ref, dinv_ref, out_ref):
    dinv = dinv_ref[...]
    out_ref[:, :128] = (a0_ref[...] + g0_ref[...]) * dinv
    out_ref[:, 128:] = (a1_ref[...] + g1_ref[...]) * dinv


_final = pl.pallas_call(
    _final_body,
    grid=(GRID,),
    in_specs=[
        pl.BlockSpec((BR, 128), lambda i: (i, 0)),
        pl.BlockSpec((BR, 128), lambda i: (i, 0)),
        pl.BlockSpec((BR, 128), lambda i: (i, 0)),
        pl.BlockSpec((BR, 128), lambda i: (i, 0)),
        pl.BlockSpec((BR, 1), lambda i: (i, 0)),
    ],
    out_specs=pl.BlockSpec((BR, 256), lambda i: (i, 0)),
    out_shape=jax.ShapeDtypeStruct((NP, 256), F32),
)


def kernel(x, edge_index, W1, W2, W3):
    ei = edge_index.astype(jnp.int32)
    pad = jnp.full((EP - E,), N, dtype=jnp.int32)
    src = jnp.concatenate([ei[0], pad])
    dst = jnp.concatenate([ei[1], pad])
    idx4 = jnp.stack([src.reshape(GROUPS, GW), dst.reshape(GROUPS, GW)],
                     axis=1).reshape(GROUPS // NBUF, NBUF, 2, GW)
    dst3d = dst.reshape(EP // CHUNK, CHUNK // 128, 128)
    xp = jnp.pad(x, ((0, NP - N), (0, 0)))

    ones_c = jnp.ones((CHUNK, 16), F32)
    zeros16 = jnp.zeros((ROWS_PER_TILE, 16), F32)
    zeros128 = jnp.zeros((ROWS_PER_TILE, 128), F32)

    deg_parts = _deg_call(dst3d, ones_c, zeros16)
    g0, g1, dinv = _prologue(xp, W1, deg_parts[0], deg_parts[1])

    a0, a1 = _agg_call(idx4, g0, g1, zeros128)
    g0, g1 = _mid(a0, a1, g0, g1, dinv, W2)

    a0, a1 = _agg_call(idx4, g0, g1, zeros128)
    g0, g1 = _mid(a0, a1, g0, g1, dinv, W3)

    a0, a1 = _agg_call(idx4, g0, g1, zeros128)
    out = _final(a0, a1, g0, g1, dinv)
    return out[:N]


# speedup vs baseline: 8.2587x; 1.0303x over previous
import functools

import jax
import jax.numpy as jnp
from jax import lax
from jax.experimental import pallas as pl
from jax.experimental.pallas import tpu as pltpu
from jax.experimental.pallas import tpu_sc as plsc

N = 10000
NP = 10240
E = 320000
EP = 327680
NC = 2
NS = 16
CHUNK = 256
ROWS_PER_TILE = NP // NS
DEG_PER_TILE = EP // (NC * NS)
DEG_CHUNKS = DEG_PER_TILE // CHUNK
GW = 32
GROUPS = EP // GW
GROUPS_PER_TILE = GROUPS // NS
F32 = jnp.float32

_mesh = plsc.VectorSubcoreMesh(core_axis_name="c", subcore_axis_name="s",
                               num_cores=NC, num_subcores=NS)
_sc_params = pltpu.CompilerParams(use_tc_tiling_on_sc=False)


def _deg_body(dst3d_hbm, ones_hbm, zeros16_hbm, out_hbm,
              idx_v, ones_v, acc, _):
    cid = lax.axis_index("c")
    sid = lax.axis_index("s")
    wid = cid * NS + sid
    pltpu.sync_copy(zeros16_hbm, acc.at[pl.ds(sid * ROWS_PER_TILE, ROWS_PER_TILE)])
    pltpu.sync_copy(ones_hbm, ones_v)
    plsc.subcore_barrier()

    def chunk(i, _):
        pltpu.sync_copy(dst3d_hbm.at[wid * DEG_CHUNKS + i], idx_v)
        for j in range(CHUNK // 128):
            pltpu.sync_copy(ones_v.at[pl.ds(j * 128, 128)],
                            acc.at[idx_v.at[j]], add=True)
        return 0

    lax.fori_loop(0, DEG_CHUNKS, chunk, 0)
    plsc.subcore_barrier()
    sl = pl.ds(sid * ROWS_PER_TILE, ROWS_PER_TILE)

    @pl.when(cid == 0)
    def _():
        pltpu.sync_copy(acc.at[sl], out_hbm.at[0].at[sl])

    @pl.when(cid == 1)
    def _():
        pltpu.sync_copy(acc.at[sl], out_hbm.at[1].at[sl])


_deg_call = functools.partial(
    pl.kernel,
    _deg_body,
    out_type=jax.ShapeDtypeStruct((NC, NP, 16), F32),
    mesh=_mesh,
    compiler_params=_sc_params,
    scratch_types=[
        pltpu.VMEM((CHUNK // 128, 128), jnp.int32),
        pltpu.VMEM((CHUNK, 16), F32),
        pltpu.VMEM_SHARED((NP, 16), F32),
        pltpu.SemaphoreType.DMA,
    ],
)()


NBUF = 8
ITERS = GROUPS_PER_TILE // NBUF


def _agg_body(idx4_hbm, g0_hbm, g1_hbm, zeros_hbm,
              out0_hbm, out1_hbm, idxA, idxB,
              r0, r1, r2, r3, r4, r5, r6, r7, acc,
              g0s, g1s, g2s, g3s, g4s, g5s, g6s, g7s,
              s0s, s1s, s2s, s3s, s4s, s5s, s6s, s7s):
    rows = [r0, r1, r2, r3, r4, r5, r6, r7]
    gsem = [g0s, g1s, g2s, g3s, g4s, g5s, g6s, g7s]
    ssem = [s0s, s1s, s2s, s3s, s4s, s5s, s6s, s7s]
    cid = lax.axis_index("c")
    sid = lax.axis_index("s")
    sl = pl.ds(sid * ROWS_PER_TILE, ROWS_PER_TILE)
    pltpu.sync_copy(zeros_hbm, acc.at[sl])
    plsc.subcore_barrier()

    def run(g_hbm):
        base = sid * ITERS
        pltpu.sync_copy(idx4_hbm.at[base], idxA)
        for b in range(NBUF):
            pltpu.async_copy(g_hbm.at[idxA.at[b].at[0]], rows[b], gsem[b])

        def phase(t_next, idx_cur, idx_nxt):
            @pl.when(t_next < ITERS)
            def _():
                pltpu.sync_copy(idx4_hbm.at[base + t_next], idx_nxt)

            for b in range(NBUF):
                pltpu.make_async_copy(g_hbm.at[idx_cur.at[b].at[0]],
                                      rows[b], gsem[b]).wait()
                pltpu.async_copy(rows[b], acc.at[idx_cur.at[b].at[1]],
                                 ssem[b], add=True)
            for b in range(NBUF):
                pltpu.make_async_copy(rows[b], acc.at[idx_cur.at[b].at[1]],
                                      ssem[b]).wait()

                @pl.when(t_next < ITERS)
                def _():
                    pltpu.async_copy(g_hbm.at[idx_nxt.at[b].at[0]],
                                     rows[b], gsem[b])

        def body(u, _):
            phase(2 * u + 1, idxA, idxB)
            phase(2 * u + 2, idxB, idxA)
            return 0

        lax.fori_loop(0, ITERS // 2, body, 0)

    @pl.when(cid == 0)
    def _():
        run(g0_hbm)

    @pl.when(cid == 1)
    def _():
        run(g1_hbm)

    plsc.subcore_barrier()

    @pl.when(cid == 0)
    def _():
        pltpu.sync_copy(acc.at[sl], out0_hbm.at[sl])

    @pl.when(cid == 1)
    def _():
        pltpu.sync_copy(acc.at[sl], out1_hbm.at[sl])


_agg_call = functools.partial(
    pl.kernel,
    _agg_body,
    out_type=(jax.ShapeDtypeStruct((NP, 128), F32),
              jax.ShapeDtypeStruct((NP, 128), F32)),
    mesh=_mesh,
    compiler_params=_sc_params,
    scratch_types=[
        pltpu.VMEM((NBUF, 2, GW), jnp.int32),
        pltpu.VMEM((NBUF, 2, GW), jnp.int32),
        pltpu.VMEM((GW, 128), F32),
        pltpu.VMEM((GW, 128), F32),
        pltpu.VMEM((GW, 128), F32),
        pltpu.VMEM((GW, 128), F32),
        pltpu.VMEM((GW, 128), F32),
        pltpu.VMEM((GW, 128), F32),
        pltpu.VMEM((GW, 128), F32),
        pltpu.VMEM((GW, 128), F32),
        pltpu.VMEM_SHARED((NP, 128), F32),
        pltpu.SemaphoreType.DMA,
        pltpu.SemaphoreType.DMA,
        pltpu.SemaphoreType.DMA,
        pltpu.SemaphoreType.DMA,
        pltpu.SemaphoreType.DMA,
        pltpu.SemaphoreType.DMA,
        pltpu.SemaphoreType.DMA,
        pltpu.SemaphoreType.DMA,
        pltpu.SemaphoreType.DMA,
        pltpu.SemaphoreType.DMA,
        pltpu.SemaphoreType.DMA,
        pltpu.SemaphoreType.DMA,
        pltpu.SemaphoreType.DMA,
        pltpu.SemaphoreType.DMA,
        pltpu.SemaphoreType.DMA,
        pltpu.SemaphoreType.DMA,
    ],
)()


BR = 1024
GRID = NP // BR
_HI = lax.Precision.HIGHEST


def _mm(a, b):
    return lax.dot_general(a, b, (((1,), (0,)), ((), ())),
                           precision=_HI, preferred_element_type=F32)


def _mmx_body(x_ref, w_ref, h0_ref, h1_ref):
    h = _mm(x_ref[...], w_ref[...])
    h0_ref[...] = h[:, :128]
    h1_ref[...] = h[:, 128:]


_mmx = pl.pallas_call(
    _mmx_body,
    grid=(GRID,),
    in_specs=[
        pl.BlockSpec((BR, 128), lambda i: (i, 0)),
        pl.BlockSpec((128, 256), lambda i: (0, 0)),
    ],
    out_specs=(
        pl.BlockSpec((BR, 128), lambda i: (i, 0)),
        pl.BlockSpec((BR, 128), lambda i: (i, 0)),
    ),
    out_shape=(
        jax.ShapeDtypeStruct((NP, 128), F32),
        jax.ShapeDtypeStruct((NP, 128), F32),
    ),
)


def _scale_body(h0_ref, h1_ref, d0_ref, d1_ref, g0_ref, g1_ref, dinv_ref):
    deg = d0_ref[:, 0:1] + d1_ref[:, 0:1] + 1.0
    dinv = lax.rsqrt(deg)
    g0_ref[...] = h0_ref[...] * dinv
    g1_ref[...] = h1_ref[...] * dinv
    dinv_ref[...] = dinv


_scale = pl.pallas_call(
    _scale_body,
    grid=(GRID,),
    in_specs=[
        pl.BlockSpec((BR, 128), lambda i: (i, 0)),
        pl.BlockSpec((BR, 128), lambda i: (i, 0)),
        pl.BlockSpec((BR, 16), lambda i: (i, 0)),
        pl.BlockSpec((BR, 16), lambda i: (i, 0)),
    ],
    out_specs=(
        pl.BlockSpec((BR, 128), lambda i: (i, 0)),
        pl.BlockSpec((BR, 128), lambda i: (i, 0)),
        pl.BlockSpec((BR, 1), lambda i: (i, 0)),
    ),
    out_shape=(
        jax.ShapeDtypeStruct((NP, 128), F32),
        jax.ShapeDtypeStruct((NP, 128), F32),
        jax.ShapeDtypeStruct((NP, 1), F32),
    ),
)


def _mid_body(a0_ref, a1_ref, g0_ref, g1_ref, dinv_ref, w_ref,
              ng0_ref, ng1_ref):
    dinv = dinv_ref[...]
    xa = jnp.maximum((a0_ref[...] + g0_ref[...]) * dinv, 0.0)
    xb = jnp.maximum((a1_ref[...] + g1_ref[...]) * dinv, 0.0)
    g = (_mm(xa, w_ref[:128, :]) + _mm(xb, w_ref[128:, :])) * dinv
    ng0_ref[...] = g[:, :128]
    ng1_ref[...] = g[:, 128:]


_mid = pl.pallas_call(
    _mid_body,
    grid=(GRID,),
    in_specs=[
        pl.BlockSpec((BR, 128), lambda i: (i, 0)),
        pl.BlockSpec((BR, 128), lambda i: (i, 0)),
        pl.BlockSpec((BR, 128), lambda i: (i, 0)),
        pl.BlockSpec((BR, 128), lambda i: (i, 0)),
        pl.BlockSpec((BR, 1), lambda i: (i, 0)),
        pl.BlockSpec((256, 256), lambda i: (0, 0)),
    ],
    out_specs=(
        pl.BlockSpec((BR, 128), lambda i: (i, 0)),
        pl.BlockSpec((BR, 128), lambda i: (i, 0)),
    ),
    out_shape=(
        jax.ShapeDtypeStruct((NP, 128), F32),
        jax.ShapeDtypeStruct((NP, 128), F32),
    ),
)


def _final_body(a0_ref, a1_ref, g0_ref, g1_ref, dinv_ref, out_ref):
    dinv = dinv_ref[...]
    out_ref[:, :128] = (a0_ref[...] + g0_ref[...]) * dinv
    out_ref[:, 128:] = (a1_ref[...] + g1_ref[...]) * dinv


_final = pl.pallas_call(
    _final_body,
    grid=(GRID,),
    in_specs=[
        pl.BlockSpec((BR, 128), lambda i: (i, 0)),
        pl.BlockSpec((BR, 128), lambda i: (i, 0)),
        pl.BlockSpec((BR, 128), lambda i: (i, 0)),
        pl.BlockSpec((BR, 128), lambda i: (i, 0)),
        pl.BlockSpec((BR, 1), lambda i: (i, 0)),
    ],
    out_specs=pl.BlockSpec((BR, 256), lambda i: (i, 0)),
    out_shape=jax.ShapeDtypeStruct((NP, 256), F32),
)


def kernel(x, edge_index, W1, W2, W3):
    ei = edge_index.astype(jnp.int32)
    pad = jnp.full((EP - E,), N, dtype=jnp.int32)
    src = jnp.concatenate([ei[0], pad])
    dst = jnp.concatenate([ei[1], pad])
    idx4 = jnp.stack([src.reshape(GROUPS, GW), dst.reshape(GROUPS, GW)],
                     axis=1).reshape(GROUPS // NBUF, NBUF, 2, GW)
    dst3d = dst.reshape(EP // CHUNK, CHUNK // 128, 128)
    xp = jnp.pad(x, ((0, NP - N), (0, 0)))

    ones_c = jnp.ones((CHUNK, 16), F32)
    zeros16 = jnp.zeros((ROWS_PER_TILE, 16), F32)
    zeros128 = jnp.zeros((ROWS_PER_TILE, 128), F32)

    h0, h1 = _mmx(xp, W1)
    deg_parts = _deg_call(dst3d, ones_c, zeros16)
    g0, g1, dinv = _scale(h0, h1, deg_parts[0], deg_parts[1])

    a0, a1 = _agg_call(idx4, g0, g1, zeros128)
    g0, g1 = _mid(a0, a1, g0, g1, dinv, W2)

    a0, a1 = _agg_call(idx4, g0, g1, zeros128)
    g0, g1 = _mid(a0, a1, g0, g1, dinv, W3)

    a0, a1 = _agg_call(idx4, g0, g1, zeros128)
    out = _final(a0, a1, g0, g1, dinv)
    return out[:N]
